# baseline (speedup 1.0000x reference)
import functools

import jax
import jax.numpy as jnp
from jax import lax
from jax.experimental import pallas as pl
from jax.experimental.pallas import tpu as pltpu
from jax.experimental.pallas import tpu_sc as plsc

N = 100000
D = 128
V = 21
NW = 32
G = 400
NG = N // G
KMAX = -(-NG // NW)
NFULL = NG - NW * (KMAX - 1)
NB = 2

_mesh = plsc.VectorSubcoreMesh(core_axis_name="c", subcore_axis_name="s")


@functools.partial(
    pl.kernel,
    mesh=_mesh,
    out_type=jax.ShapeDtypeStruct((N * D,), jnp.float32),
    compiler_params=pltpu.CompilerParams(needs_layout_passes=False),
    scratch_types=[
        pltpu.VMEM((V * D,), jnp.float32),
        pltpu.VMEM((G * D,), jnp.float32),
        pltpu.VMEM((G * D,), jnp.float32),
        pltpu.VMEM((KMAX * G,), jnp.int32),
        pltpu.SemaphoreType.DMA,
        pltpu.SemaphoreType.DMA,
    ],
)
def _sc_embed(table_hbm, idx_hbm, out_hbm, table_v, buf0, buf1, idx_v, s0, s1):
    wid = lax.axis_index("s") * 2 + lax.axis_index("c")
    full = wid < NFULL
    a = jnp.where(full, KMAX * wid, (KMAX - 1) * wid + NFULL)
    ng = jnp.where(full, KMAX, KMAX - 1)

    pltpu.sync_copy(table_hbm, table_v)

    base = pl.multiple_of(G * a, 8)
    pltpu.sync_copy(idx_hbm.at[pl.ds(base, (KMAX - 1) * G)],
                    idx_v.at[pl.ds(0, (KMAX - 1) * G)])
    last = pl.multiple_of(G * (a + ng - 1), 8)
    pltpu.sync_copy(idx_hbm.at[pl.ds(last, G)],
                    idx_v.at[pl.ds((KMAX - 1) * G, G)])

    ssems = (s0, s1)
    bufs = (buf0, buf1)
    lane = lax.broadcasted_iota(jnp.int32, (16,), 0)

    def out_slice(k):
        gk = jnp.minimum(k, ng - 1)
        return out_hbm.at[pl.ds(pl.multiple_of((a + gk) * (G * D), 8), G * D)]

    def build(k, buf):
        iofs = G * jnp.minimum(k, ng - 1)

        @plsc.parallel_loop(0, G // 16, 1, unroll=8)
        def tile(t):
            iv = idx_v[pl.ds(iofs + 16 * t, 16)]
            rbase = 16 * t * D
            for l in range(16):
                sb = iv[l] * D
                vals = [table_v[pl.ds(sb + 16 * j, 16)] for j in range(8)]
                for j, v in enumerate(vals):
                    buf[pl.ds(rbase + l * D + 16 * j, 16)] = v

    def super_it(m, _):
        for b in range(NB):
            k = m * NB + b

            @pl.when(m > 0)
            def _():
                pltpu.make_async_copy(bufs[b], out_slice(k - NB),
                                      ssems[b]).wait()

            build(k, bufs[b])
            pltpu.async_copy(bufs[b], out_slice(k), ssems[b])
        return 0

    lax.fori_loop(0, KMAX // NB, super_it, 0)
    for b in range(NB):
        pltpu.make_async_copy(bufs[b], out_slice(KMAX - NB + b),
                              ssems[b]).wait()


def kernel(x, enc_weight):
    idx = x.reshape(N).astype(jnp.int32)
    out = _sc_embed(enc_weight.reshape(V * D), idx)
    return out.reshape(N, D)


# speedup vs baseline: 1.7589x; 1.7589x over previous
import functools

import jax
import jax.numpy as jnp
from jax import lax
from jax.experimental import pallas as pl
from jax.experimental.pallas import tpu as pltpu
from jax.experimental.pallas import tpu_sc as plsc

N = 100000
D = 128
V = 21
NW = 32
G = 400
NG = N // G
KMAX = -(-NG // NW)
NFULL = NG - NW * (KMAX - 1)
NB = 2

_mesh = plsc.VectorSubcoreMesh(core_axis_name="c", subcore_axis_name="s")


@functools.partial(
    pl.kernel,
    mesh=_mesh,
    out_type=jax.ShapeDtypeStruct((N * D,), jnp.float32),
    compiler_params=pltpu.CompilerParams(needs_layout_passes=False),
    scratch_types=[
        pltpu.VMEM((V * D,), jnp.float32),
        pltpu.VMEM((G * D,), jnp.float32),
        pltpu.VMEM((G * D,), jnp.float32),
        pltpu.VMEM((KMAX * G,), jnp.int32),
        pltpu.SemaphoreType.DMA,
        pltpu.SemaphoreType.DMA,
    ],
)
def _sc_embed(table_hbm, idx_hbm, out_hbm, table_v, buf0, buf1, idx_v, s0, s1):
    wid = lax.axis_index("s") * 2 + lax.axis_index("c")
    full = wid < NFULL
    a = jnp.where(full, KMAX * wid, (KMAX - 1) * wid + NFULL)
    ng = jnp.where(full, KMAX, KMAX - 1)

    pltpu.sync_copy(table_hbm, table_v)

    base = pl.multiple_of(G * a, 8)
    pltpu.sync_copy(idx_hbm.at[pl.ds(base, (KMAX - 1) * G)],
                    idx_v.at[pl.ds(0, (KMAX - 1) * G)])
    last = pl.multiple_of(G * (a + ng - 1), 8)
    pltpu.sync_copy(idx_hbm.at[pl.ds(last, G)],
                    idx_v.at[pl.ds((KMAX - 1) * G, G)])

    ssems = (s0, s1)
    bufs = (buf0, buf1)
    lane = lax.broadcasted_iota(jnp.int32, (16,), 0)

    def out_slice(k):
        gk = jnp.minimum(k, ng - 1)
        return out_hbm.at[pl.ds(pl.multiple_of((a + gk) * (G * D), 8), G * D)]

    def build(k, buf):
        iofs = G * jnp.minimum(k, ng - 1)

        @plsc.parallel_loop(0, G // 16, 1, unroll=5)
        def tile(t):
            iv = idx_v[pl.ds(iofs + 16 * t, 16)]
            rbase = 16 * t * D
            for l in range(16):
                sb = iv[l] * D
                vals = [table_v[pl.ds(sb + 16 * j, 16)] for j in range(8)]
                for j, v in enumerate(vals):
                    buf[pl.ds(rbase + l * D + 16 * j, 16)] = v

    def super_it(m, _):
        for b in range(NB):
            k = m * NB + b

            @pl.when(m > 0)
            def _():
                pltpu.make_async_copy(bufs[b], out_slice(k - NB),
                                      ssems[b]).wait()

            build(k, bufs[b])
            pltpu.async_copy(bufs[b], out_slice(k), ssems[b])
        return 0

    lax.fori_loop(0, KMAX // NB, super_it, 0)
    for b in range(NB):
        pltpu.make_async_copy(bufs[b], out_slice(KMAX - NB + b),
                              ssems[b]).wait()


def kernel(x, enc_weight):
    idx = x.reshape(N).astype(jnp.int32)
    out = _sc_embed(enc_weight.reshape(V * D), idx)
    return out.reshape(N, D)
